# baseline (speedup 1.0000x reference)
import functools

import jax
import jax.numpy as jnp
from jax import lax
from jax.experimental import pallas as pl
from jax.experimental.pallas import tpu as pltpu
from jax.experimental.pallas import tpu_sc as plsc

_L = 16
_N = 8192
_C = _N // _L
_CS = _C + 1
_R = 2048
_HEADS_PER_WORKER = 2


def _keys_tc(q0, kt):
  def body(q_ref, k_ref, o_ref):
    for h in range(2):
      s = lax.dot_general(q_ref[0, h:h + 1, :], k_ref[0, h],
                          (((1,), (0,)), ((), ())),
                          preferred_element_type=jnp.float32)
      col = lax.broadcasted_iota(jnp.int32, s.shape, 1)
      s = jnp.where(col == 0, jnp.float32(jnp.inf), s)
      b = lax.bitcast_convert_type(s, jnp.int32)
      m = lax.shift_right_arithmetic(b, 31)
      keys = b ^ (m | jnp.int32(-2147483648))
      o_ref[0, h] = keys.reshape(_N // 128, 128)

  g = q0.shape[0]
  return pl.pallas_call(
      body,
      grid=(g,),
      in_specs=[
          pl.BlockSpec((1, 2, 64), lambda i: (i, 0, 0)),
          pl.BlockSpec((1, 2, 64, _N), lambda i: (i, 0, 0, 0)),
      ],
      out_specs=pl.BlockSpec((1, 2, _N // 128, 128), lambda i: (i, 0, 0, 0)),
      out_shape=jax.ShapeDtypeStruct((g, 2, _N // 128, 128), jnp.int32),
  )(q0, kt)


def _mask_sc(keys):
  nheads = keys.shape[0]
  hpw = nheads // 32
  mesh = plsc.VectorSubcoreMesh(
      core_axis_name="c", subcore_axis_name="s", num_cores=2, num_subcores=16)

  @functools.partial(
      pl.kernel,
      out_type=jax.ShapeDtypeStruct((nheads, _N // 128, 128), jnp.int32),
      mesh=mesh,
      compiler_params=pltpu.CompilerParams(needs_layout_passes=False),
      scratch_types=[
          pltpu.VMEM((_N // 128, 128), jnp.int32),
          pltpu.VMEM((_L * _CS,), jnp.int32),
          pltpu.VMEM((_L * _CS,), jnp.int32),
          pltpu.VMEM((_L * _CS,), jnp.int32),
          pltpu.VMEM((_L * _CS,), jnp.int32),
          pltpu.VMEM((_N // 128, 128), jnp.int32),
          pltpu.VMEM((256 * _L,), jnp.int32),
      ],
  )
  def sc_kernel(keys_hbm, out_hbm, stage, ks1, ks2, fs1, fs2, ob, hist):
    num_cores = 2
    wid = lax.axis_index("s") * num_cores + lax.axis_index("c")
    lane = lax.iota(jnp.int32, _L)
    base_lin = lane * _C
    base_skw = lane * _CS
    zeros16 = jnp.zeros((_L,), jnp.int32)
    ones16 = jnp.ones((_L,), jnp.int32)

    def zero_hist():
      def zero_body(i, _):
        hist[pl.ds(i * _L, _L)] = zeros16
        return 0

      lax.fori_loop(0, 256, zero_body, 0, unroll=8)

    def scan_hist():
      def scan_body(i, carry):
        hv = hist[pl.ds(i * _L, _L)]
        incl = plsc.cumsum(hv)
        hist[pl.ds(i * _L, _L)] = incl - hv + carry
        return carry + jnp.max(incl)

      lax.fori_loop(0, 256, scan_body, jnp.int32(0), unroll=2)

    for n in range(hpw):
      h = wid * hpw + n
      pltpu.sync_copy(keys_hbm.at[h], stage)

      zero_hist()

      def hist0_body(t, _):
        j = base_lin + t
        kv = plsc.load_gather(stage, [j >> 7, j & 127])
        plsc.addupdate_scatter(hist, [((kv & 255) << 4) | lane], ones16)
        return 0

      lax.fori_loop(0, _C, hist0_body, 0, unroll=4)
      scan_hist()

      def perm0_body(t, _):
        j = base_lin + t
        kv = plsc.load_gather(stage, [j >> 7, j & 127])
        fv = (j <= _R).astype(jnp.int32)
        a = ((kv & 255) << 4) | lane
        off = plsc.load_gather(hist, [a])
        plsc.store_scatter(hist, [a], off + 1)
        sk = off + (off >> 9)
        plsc.store_scatter(ks1, [sk], kv)
        plsc.store_scatter(fs1, [sk], fv)
        return 0

      lax.fori_loop(0, _C, perm0_body, 0, unroll=2)

      plan = [(ks1, fs1, ks2, fs2), (ks2, fs2, ks1, fs1),
              (ks1, fs1, None, None)]
      for p, (src_k, src_f, dst_k, dst_f) in enumerate(plan, start=1):
        shift = p * 8
        zero_hist()

        def hist_body(t, _, src_k=src_k, shift=shift):
          kv = plsc.load_gather(src_k, [base_skw + t])
          d = (kv >> shift) & 255
          plsc.addupdate_scatter(hist, [(d << 4) | lane], ones16)
          return 0

        lax.fori_loop(0, _C, hist_body, 0, unroll=4)
        scan_hist()

        if p < 3:

          def perm_body(t, _, src_k=src_k, src_f=src_f, dst_k=dst_k,
                        dst_f=dst_f, shift=shift):
            idx = base_skw + t
            kv = plsc.load_gather(src_k, [idx])
            fv = plsc.load_gather(src_f, [idx])
            a = (((kv >> shift) & 255) << 4) | lane
            off = plsc.load_gather(hist, [a])
            plsc.store_scatter(hist, [a], off + 1)
            sk = off + (off >> 9)
            plsc.store_scatter(dst_k, [sk], kv)
            plsc.store_scatter(dst_f, [sk], fv)
            return 0

        else:

          def perm_body(t, _, src_k=src_k, src_f=src_f, shift=shift):
            idx = base_skw + t
            kv = plsc.load_gather(src_k, [idx])
            fv = plsc.load_gather(src_f, [idx])
            a = (((kv >> shift) & 255) << 4) | lane
            off = plsc.load_gather(hist, [a])
            plsc.store_scatter(hist, [a], off + 1)
            tgt = (off + 1) & (_N - 1)
            plsc.store_scatter(ob, [tgt >> 7, tgt & 127], fv)
            return 0

        lax.fori_loop(0, _C, perm_body, 0, unroll=2)

      pltpu.sync_copy(ob, out_hbm.at[h])

  return sc_kernel(keys)


def kernel(q, k):
  b, nh, s, d = q.shape
  q0 = q[:, :, 0, :].reshape(32, 2, d)
  kt = jnp.swapaxes(k, -1, -2).reshape(32, 2, d, s)
  masks = []
  for c in range(2):
    keys_c = _keys_tc(q0[c * 16:(c + 1) * 16], kt[c * 16:(c + 1) * 16])
    masks.append(_mask_sc(keys_c.reshape(32, s // 128, 128)))
  mask = jnp.concatenate(masks)
  return (mask != 0).reshape(b, nh, s)


# speedup vs baseline: 1.1104x; 1.1104x over previous
import functools

import jax
import jax.numpy as jnp
from jax import lax
from jax.experimental import pallas as pl
from jax.experimental.pallas import tpu as pltpu
from jax.experimental.pallas import tpu_sc as plsc

_L = 16
_N = 8192
_C = _N // _L
_CS = _C + 1
_R = 2048
_HEADS_PER_WORKER = 2


def _keys_tc(q0, kt):
  def body(q_ref, k_ref, o_ref):
    for h in range(2):
      s = lax.dot_general(q_ref[0, h:h + 1, :], k_ref[0, h],
                          (((1,), (0,)), ((), ())),
                          preferred_element_type=jnp.float32)
      col = lax.broadcasted_iota(jnp.int32, s.shape, 1)
      s = jnp.where(col == 0, jnp.float32(jnp.inf), s)
      b = lax.bitcast_convert_type(s, jnp.int32)
      m = lax.shift_right_arithmetic(b, 31)
      keys = b ^ (m | jnp.int32(-2147483648))
      o_ref[0, h] = keys.reshape(_N // 128, 128)

  g = q0.shape[0]
  return pl.pallas_call(
      body,
      grid=(g,),
      in_specs=[
          pl.BlockSpec((1, 2, 64), lambda i: (i, 0, 0)),
          pl.BlockSpec((1, 2, 64, _N), lambda i: (i, 0, 0, 0)),
      ],
      out_specs=pl.BlockSpec((1, 2, _N // 128, 128), lambda i: (i, 0, 0, 0)),
      out_shape=jax.ShapeDtypeStruct((g, 2, _N // 128, 128), jnp.int32),
  )(q0, kt)


def _mask_sc(keys):
  nheads = keys.shape[0]
  hpw = nheads // 32
  mesh = plsc.VectorSubcoreMesh(
      core_axis_name="c", subcore_axis_name="s", num_cores=2, num_subcores=16)

  @functools.partial(
      pl.kernel,
      out_type=jax.ShapeDtypeStruct((nheads, _N // 128, 128), jnp.int32),
      mesh=mesh,
      compiler_params=pltpu.CompilerParams(needs_layout_passes=False),
      scratch_types=[
          pltpu.VMEM((_N // 128, 128), jnp.int32),
          pltpu.VMEM((_L * _CS,), jnp.int32),
          pltpu.VMEM((_L * _CS,), jnp.int32),
          pltpu.VMEM((_L * _CS,), jnp.int32),
          pltpu.VMEM((_L * _CS,), jnp.int32),
          pltpu.VMEM((_N // 128, 128), jnp.int32),
          pltpu.VMEM((256 * _L,), jnp.int32),
      ],
  )
  def sc_kernel(keys_hbm, out_hbm, stage, ks1, ks2, fs1, fs2, ob, hist):
    num_cores = 2
    wid = lax.axis_index("s") * num_cores + lax.axis_index("c")
    lane = lax.iota(jnp.int32, _L)
    base_lin = lane * _C
    base_skw = lane * _CS
    zeros16 = jnp.zeros((_L,), jnp.int32)
    ones16 = jnp.ones((_L,), jnp.int32)

    def zero_hist():
      def zero_body(i, _):
        hist[pl.ds(i * _L, _L)] = zeros16
        return 0

      lax.fori_loop(0, 256, zero_body, 0, unroll=8)

    def scan_hist():
      def scan_body(i, carry):
        hv = hist[pl.ds(i * _L, _L)]
        incl = plsc.cumsum(hv)
        hist[pl.ds(i * _L, _L)] = incl - hv + carry
        return carry + jnp.max(incl)

      lax.fori_loop(0, 256, scan_body, jnp.int32(0), unroll=2)

    for n in range(hpw):
      h = wid * hpw + n
      pltpu.sync_copy(keys_hbm.at[h], stage)

      zero_hist()

      def hist0_body(t, _):
        j = base_lin + t
        kv = plsc.load_gather(stage, [j >> 7, j & 127])
        plsc.addupdate_scatter(hist, [((kv & 255) << 4) | lane], ones16)
        return 0

      lax.fori_loop(0, _C, hist0_body, 0, unroll=4)
      scan_hist()

      def perm0_body(t, _):
        j = base_lin + t
        kv = plsc.load_gather(stage, [j >> 7, j & 127])
        fv = (j <= _R).astype(jnp.int32)
        a = ((kv & 255) << 4) | lane
        off = plsc.load_gather(hist, [a])
        plsc.store_scatter(hist, [a], off + 1)
        sk = off + (off >> 9)
        plsc.store_scatter(ks1, [sk], kv)
        plsc.store_scatter(fs1, [sk], fv)
        return 0

      lax.fori_loop(0, _C, perm0_body, 0, unroll=2)

      plan = [(ks1, fs1, ks2, fs2), (ks2, fs2, ks1, fs1),
              (ks1, fs1, None, None)]
      for p, (src_k, src_f, dst_k, dst_f) in enumerate(plan, start=1):
        shift = p * 8
        zero_hist()

        def hist_body(t, _, src_k=src_k, shift=shift):
          kv = plsc.load_gather(src_k, [base_skw + t])
          d = (kv >> shift) & 255
          plsc.addupdate_scatter(hist, [(d << 4) | lane], ones16)
          return 0

        lax.fori_loop(0, _C, hist_body, 0, unroll=4)
        scan_hist()

        if p < 3:

          def perm_body(t, _, src_k=src_k, src_f=src_f, dst_k=dst_k,
                        dst_f=dst_f, shift=shift):
            idx = base_skw + t
            kv = plsc.load_gather(src_k, [idx])
            fv = plsc.load_gather(src_f, [idx])
            a = (((kv >> shift) & 255) << 4) | lane
            off = plsc.load_gather(hist, [a])
            plsc.store_scatter(hist, [a], off + 1)
            sk = off + (off >> 9)
            plsc.store_scatter(dst_k, [sk], kv)
            plsc.store_scatter(dst_f, [sk], fv)
            return 0

        else:

          def perm_body(t, _, src_k=src_k, src_f=src_f, shift=shift):
            idx = base_skw + t
            kv = plsc.load_gather(src_k, [idx])
            fv = plsc.load_gather(src_f, [idx])
            a = (((kv >> shift) & 255) << 4) | lane
            off = plsc.load_gather(hist, [a])
            plsc.store_scatter(hist, [a], off + 1)
            tgt = (off + 1) & (_N - 1)
            plsc.store_scatter(ob, [tgt >> 7, tgt & 127], fv)
            return 0

        lax.fori_loop(0, _C, perm_body, 0, unroll=2)

      pltpu.sync_copy(ob, out_hbm.at[h])

  return sc_kernel(keys)


def kernel(q, k):
  b, nh, s, d = q.shape
  q0 = q[:, :, 0, :].reshape(32, 2, d)
  kt = jnp.swapaxes(k, -1, -2).reshape(32, 2, d, s)
  keys = _keys_tc(q0, kt)
  mask = _mask_sc(keys.reshape(b * nh, s // 128, 128))
  return (mask != 0).reshape(b, nh, s)
